# baseline (speedup 1.0000x reference)
import functools

import jax
import jax.numpy as jnp
import numpy as np
from jax import lax
from jax.experimental import pallas as pl
from jax.experimental.pallas import tpu as pltpu
from jax.experimental.pallas import tpu_sc as plsc

B, T, D = 64, 8192, 16
R = B * D
NC, NS = 2, 16
NW = NC * NS
TGS = 8
NTG = NW // TGS
BPG = B // NTG
CW = T // TGS
RB = D
NBLK = BPG
G = 16

MAGIC = np.float32(12582912.0)

_S = [-1.0, 0.0, -5.0, 0.0]
_E = [1.0, 10.0, 5.0, 1.0]
_SZ = [256.0, 1024.0, 512.0, 2.0]
_IR = [float(np.float32(1.0) / (np.float32(e - s) / np.float32(sz - 1.0)))
       for s, e, sz in zip(_S, _E, _SZ)]
_S_TAB = tuple(_S + [0.0] * 12)
_E_TAB = tuple(_E + [1.0] * 12)
_IR_TAB = tuple(_IR + [1.0] * 12)
_TABS = np.stack([_S_TAB, _E_TAB, _IR_TAB]).astype(np.float32)

_mesh = plsc.VectorSubcoreMesh(
    core_axis_name="c", subcore_axis_name="s", num_cores=NC, num_subcores=NS)


@functools.partial(
    pl.kernel,
    mesh=_mesh,
    out_type=jax.ShapeDtypeStruct((R, T), jnp.float32),
    scratch_types=[
        pltpu.VMEM((BPG, CW), jnp.int32),
        pltpu.VMEM((RB, CW), jnp.float32),
        pltpu.VMEM((RB, CW), jnp.float32),
        pltpu.VMEM((RB, CW), jnp.float32),
        pltpu.VMEM((RB, CW), jnp.float32),
        pltpu.VMEM((3, G), jnp.float32),
        pltpu.SemaphoreType.DMA,
        pltpu.SemaphoreType.DMA,
        pltpu.SemaphoreType.DMA,
        pltpu.SemaphoreType.DMA,
    ],
    compiler_params=pltpu.CompilerParams(
        needs_layout_passes=False, use_tc_tiling_on_sc=True),
)
def _encode(x_hbm, ids_hbm, tabs_hbm, out_hbm, idv, xv0, xv1, ov0, ov1,
            tabv, si0, si1, so0, so1):
    wid = lax.axis_index("s") * NC + lax.axis_index("c")
    tg = wid // TGS
    tw = wid % TGS
    row0 = tg * BPG * D
    col0 = tw * CW
    pltpu.sync_copy(tabs_hbm, tabv)
    pltpu.sync_copy(ids_hbm.at[pl.ds(tg * BPG, BPG), pl.ds(col0, CW)], idv)

    s_tab = tabv[0, :]
    e_tab = tabv[1, :]
    ir_tab = tabv[2, :]

    def in_copy(blk, buf, sem):
        return pltpu.make_async_copy(
            x_hbm.at[pl.ds(row0 + blk * RB, RB), pl.ds(col0, CW)], buf, sem)

    def out_copy(blk, buf, sem):
        return pltpu.make_async_copy(
            buf, out_hbm.at[pl.ds(row0 + blk * RB, RB), pl.ds(col0, CW)],
            sem)

    NG = CW // G

    def compute(blk, xvb, ovb):
        @plsc.parallel_loop(0, NG, unroll=4)
        def _(g):
            ids16 = idv[blk, pl.ds(g * G, G)]
            s16 = s_tab.at[ids16].get(mode="promise_in_bounds")
            e16 = e_tab.at[ids16].get(mode="promise_in_bounds")
            r16 = ir_tab.at[ids16].get(mode="promise_in_bounds")
            keep = ids16 == 3
            xs = [xvb[c, pl.ds(g * G, G)] for c in range(D)]
            q = [jnp.maximum(x, s16) for x in xs]
            q = [jnp.minimum(v, e16) for v in q]
            q = [v - s16 for v in q]
            q = [v * r16 for v in q]
            q = [v + MAGIC for v in q]
            q = [v - MAGIC for v in q]
            q = [jnp.where(keep, x, v) for x, v in zip(xs, q)]
            for c in range(D):
                ovb[c, pl.ds(g * G, G)] = q[c]

    in_copy(0, xv0, si0).start()
    in_copy(1, xv1, si1).start()

    def pair_body(p, carry):
        blk0 = 2 * p
        blk1 = 2 * p + 1

        @pl.when(p > 0)
        def _():
            out_copy(blk0, ov0, so0).wait()

        in_copy(blk0, xv0, si0).wait()
        compute(blk0, xv0, ov0)
        out_copy(blk0, ov0, so0).start()

        @pl.when(blk0 + 2 < NBLK)
        def _():
            in_copy(blk0 + 2, xv0, si0).start()

        @pl.when(p > 0)
        def _():
            out_copy(blk1, ov1, so1).wait()

        in_copy(blk1, xv1, si1).wait()
        compute(blk1, xv1, ov1)
        out_copy(blk1, ov1, so1).start()

        @pl.when(blk1 + 2 < NBLK)
        def _():
            in_copy(blk1 + 2, xv1, si1).start()

        return carry

    lax.fori_loop(0, NBLK // 2, pair_body, 0)
    out_copy(NBLK - 2, ov0, so0).wait()
    out_copy(NBLK - 1, ov1, so1).wait()


def kernel(tks, tk_ids):
    xt = jnp.transpose(tks.astype(jnp.float32), (0, 2, 1)).reshape(R, T)
    out = _encode(xt, tk_ids, jnp.asarray(_TABS))
    return jnp.transpose(out.reshape(B, D, T), (0, 2, 1))


# speedup vs baseline: 1.4960x; 1.4960x over previous
import functools

import jax
import jax.numpy as jnp
import numpy as np
from jax import lax
from jax.experimental import pallas as pl
from jax.experimental.pallas import tpu as pltpu
from jax.experimental.pallas import tpu_sc as plsc

B, T, D = 64, 8192, 16
R = B * D
NC, NS = 2, 16
NW = NC * NS
TGS = 8
NTG = NW // TGS
BPG = B // NTG
CW = T // TGS
RB = D
NBLK = BPG
G = 16

MAGIC = np.float32(12582912.0)

_S = [-1.0, 0.0, -5.0, 0.0]
_E = [1.0, 10.0, 5.0, 1.0]
_SZ = [256.0, 1024.0, 512.0, 2.0]
_IR = [float(np.float32(1.0) / (np.float32(e - s) / np.float32(sz - 1.0)))
       for s, e, sz in zip(_S, _E, _SZ)]
_S_TAB = tuple(_S + [0.0] * 12)
_E_TAB = tuple(_E + [1.0] * 12)
_IR_TAB = tuple(_IR + [1.0] * 12)
_TABS = np.stack([_S_TAB, _E_TAB, _IR_TAB]).astype(np.float32)

_mesh = plsc.VectorSubcoreMesh(
    core_axis_name="c", subcore_axis_name="s", num_cores=NC, num_subcores=NS)


@functools.partial(
    pl.kernel,
    mesh=_mesh,
    out_type=jax.ShapeDtypeStruct((R, T), jnp.float32),
    scratch_types=[
        pltpu.VMEM((BPG, CW), jnp.int32),
        pltpu.VMEM((RB, CW), jnp.float32),
        pltpu.VMEM((RB, CW), jnp.float32),
        pltpu.VMEM((RB, CW), jnp.float32),
        pltpu.VMEM((RB, CW), jnp.float32),
        pltpu.VMEM((3, G), jnp.float32),
        pltpu.SemaphoreType.DMA,
        pltpu.SemaphoreType.DMA,
        pltpu.SemaphoreType.DMA,
        pltpu.SemaphoreType.DMA,
    ],
    compiler_params=pltpu.CompilerParams(
        needs_layout_passes=False, use_tc_tiling_on_sc=True),
)
def _encode(x_hbm, ids_hbm, tabs_hbm, out_hbm, idv, xv0, xv1, ov0, ov1,
            tabv, si0, si1, so0, so1):
    wid = lax.axis_index("s") * NC + lax.axis_index("c")
    tg = wid // TGS
    tw = wid % TGS
    row0 = tg * BPG * D
    col0 = tw * CW
    pltpu.sync_copy(tabs_hbm, tabv)
    pltpu.sync_copy(ids_hbm.at[pl.ds(tg * BPG, BPG), pl.ds(col0, CW)], idv)

    s_tab = tabv[0, :]
    e_tab = tabv[1, :]
    ir_tab = tabv[2, :]

    def in_copy(blk, buf, sem):
        return pltpu.make_async_copy(
            x_hbm.at[pl.ds(row0 + blk * RB, RB), pl.ds(col0, CW)], buf, sem)

    def out_copy(blk, buf, sem):
        return pltpu.make_async_copy(
            buf, out_hbm.at[pl.ds(row0 + blk * RB, RB), pl.ds(col0, CW)],
            sem)

    NG = CW // G

    def compute(blk, xvb, ovb):
        @plsc.parallel_loop(0, NG, unroll=2)
        def _(g):
            ids16 = idv[blk, pl.ds(g * G, G)]
            s16 = s_tab.at[ids16].get(mode="promise_in_bounds")
            e16 = e_tab.at[ids16].get(mode="promise_in_bounds")
            r16 = ir_tab.at[ids16].get(mode="promise_in_bounds")
            keep = ids16 == 3
            xs = [xvb[c, pl.ds(g * G, G)] for c in range(D)]
            q = [jnp.maximum(x, s16) for x in xs]
            q = [jnp.minimum(v, e16) for v in q]
            q = [v - s16 for v in q]
            q = [v * r16 for v in q]
            q = [v + MAGIC for v in q]
            q = [v - MAGIC for v in q]
            q = [jnp.where(keep, x, v) for x, v in zip(xs, q)]
            for c in range(D):
                ovb[c, pl.ds(g * G, G)] = q[c]

    in_copy(0, xv0, si0).start()
    in_copy(1, xv1, si1).start()

    def pair_body(p, carry):
        blk0 = 2 * p
        blk1 = 2 * p + 1

        @pl.when(p > 0)
        def _():
            out_copy(blk0, ov0, so0).wait()

        in_copy(blk0, xv0, si0).wait()
        compute(blk0, xv0, ov0)
        out_copy(blk0, ov0, so0).start()

        @pl.when(blk0 + 2 < NBLK)
        def _():
            in_copy(blk0 + 2, xv0, si0).start()

        @pl.when(p > 0)
        def _():
            out_copy(blk1, ov1, so1).wait()

        in_copy(blk1, xv1, si1).wait()
        compute(blk1, xv1, ov1)
        out_copy(blk1, ov1, so1).start()

        @pl.when(blk1 + 2 < NBLK)
        def _():
            in_copy(blk1 + 2, xv1, si1).start()

        return carry

    lax.fori_loop(0, NBLK // 2, pair_body, 0)
    out_copy(NBLK - 2, ov0, so0).wait()
    out_copy(NBLK - 1, ov1, so1).wait()


def kernel(tks, tk_ids):
    xt = jnp.transpose(tks.astype(jnp.float32), (0, 2, 1)).reshape(R, T)
    out = _encode(xt, tk_ids, jnp.asarray(_TABS))
    return jnp.transpose(out.reshape(B, D, T), (0, 2, 1))
